# baseline (speedup 1.0000x reference)
import functools

import numpy as np

import jax
import jax.numpy as jnp
from jax import lax
from jax.experimental import pallas as pl
from jax.experimental.pallas import tpu as pltpu
from jax.experimental.pallas import tpu_sc as plsc

B, L = 1024, 169
TILE_CLASSES, WIDTH, HEIGHT, H, O = 32, 13, 13, 128, 256
NCOMB = TILE_CLASSES * WIDTH * HEIGHT

NC, NS = 2, 16
NL, NB = 4, 8
BC = B // NB
LG = 44
_L_STARTS = (0, 43, 85, 127)
_L_SIZES = (43, 42, 42, 42)

def _ln_gelu(t, g, b):
    mu = jnp.mean(t, axis=-1, keepdims=True)
    var = jnp.mean((t - mu) ** 2, axis=-1, keepdims=True)
    v = (t - mu) / jnp.sqrt(var + 1e-5) * g + b
    return 0.5 * v * (1.0 + lax.erf(v * (2.0 ** -0.5)))


def _prep_body(tile_ref, x_ref, y_ref, tt_ref, tg_ref, tb_ref, ct_ref, cg_ref,
               cb_ref, rt_ref, rg_ref, rb_ref, w_ref, bias_ref, c_ref, idx_ref):
    f32 = jnp.float32
    pt = jnp.dot(_ln_gelu(tt_ref[...], tg_ref[...], tb_ref[...]),
                 w_ref[0:H, :], preferred_element_type=f32)
    pc = jnp.dot(_ln_gelu(ct_ref[...], cg_ref[...], cb_ref[...]),
                 w_ref[H:2 * H, :], preferred_element_type=f32)
    pr = jnp.dot(_ln_gelu(rt_ref[...], rg_ref[...], rb_ref[...]),
                 w_ref[2 * H:3 * H, :], preferred_element_type=f32)

    def onehot(nrows, ncols, row_to_col):
        r = lax.broadcasted_iota(jnp.int32, (nrows, ncols), 0)
        c = lax.broadcasted_iota(jnp.int32, (nrows, ncols), 1)
        return (row_to_col(r) == c).astype(f32)

    oht = onehot(NCOMB, TILE_CLASSES, lambda r: r // (WIDTH * HEIGHT))
    ohc = onehot(NCOMB, WIDTH, lambda r: (r // HEIGHT) % WIDTH)
    ohr = onehot(NCOMB, HEIGHT, lambda r: r % HEIGHT)
    cc = (jnp.dot(oht, pt, preferred_element_type=f32)
          + jnp.dot(ohc, pc, preferred_element_type=f32)
          + jnp.dot(ohr, pr, preferred_element_type=f32)
          + bias_ref[...])

    def bfbits(v):
        bits = lax.bitcast_convert_type(v, jnp.int32)
        return lax.shift_right_logical(
            bits + 0x7FFF + ((bits >> 16) & 1), 16)

    c_ref[...] = bfbits(cc[:, 0:H]) | (bfbits(cc[:, H:2 * H]) << 16)

    idx_t = jnp.transpose(tile_ref[...] * (WIDTH * HEIGHT)
                          + x_ref[...] * HEIGHT + y_ref[...])
    for g in range(NL):
        ls, n = _L_STARTS[g], _L_SIZES[g]
        idx_ref[g, 0:n, :] = idx_t[ls:ls + n, :]
        idx_ref[g, n:LG, :] = jnp.zeros((LG - n, B), jnp.int32)


def _prep(tile, x, y, tt, tg, tb, ct, cg, cb, rt, rg, rb, w, bias):
    return pl.pallas_call(
        _prep_body,
        out_shape=(
            jax.ShapeDtypeStruct((NCOMB, O // 2), jnp.int32),
            jax.ShapeDtypeStruct((NL, LG, B), jnp.int32),
        ),
    )(tile, x, y, tt, tg.reshape(1, H), tb.reshape(1, H), ct,
      cg.reshape(1, H), cb.reshape(1, H), rt, rg.reshape(1, H),
      rb.reshape(1, H), w, bias.reshape(1, O))


def _sc_body(idx_hbm, c_hbm, out_hbm, idxw, fb0, fb1, bb0, bb1, g0, g1, s0, s1):
    fbufs, bbufs, gsems, ssems = (fb0, fb1), (bb0, bb1), (g0, g1), (s0, s1)
    wid = lax.axis_index("s") * NC + lax.axis_index("c")
    g = wid // NB
    b0 = (wid % NB) * BC
    l_start = jnp.where(g == 0, 0, 43 + (g - 1) * 42)
    n_l = jnp.where(g == 0, 43, 42)

    pltpu.sync_copy(idx_hbm.at[g, :, pl.ds(b0, BC)], idxw)

    def g_desc(j, b):
        return pltpu.make_async_copy(c_hbm.at[idxw.at[j]], bbufs[b], gsems[b])

    def s_desc(j, b):
        return pltpu.make_async_copy(
            fbufs[b], out_hbm.at[l_start + j, pl.ds(b0, BC)], ssems[b])

    def widen(b):
        def body(t2, _):
            for dt in range(2):
                t = t2 * 2 + dt
                for k in range(O // 32):
                    xi = bbufs[b][t, pl.ds(16 * k, 16)]
                    lo = plsc.bitcast(xi << 16, jnp.float32)
                    hi = plsc.bitcast(xi & jnp.int32(-65536), jnp.float32)
                    fbufs[b][t, pl.ds(16 * k, 16)] = lo
                    fbufs[b][t, pl.ds(H + 16 * k, 16)] = hi
            return _

        lax.fori_loop(0, BC // 2, body, None)

    g_desc(0, 0).start()

    def outer(k, _):
        for b in range(2):
            j = k * 2 + b

            @pl.when(j + 1 < n_l)
            def _prefetch():
                g_desc(j + 1, 1 - b).start()

            @pl.when((j >= 2) & (j - 2 < n_l))
            def _retire():
                s_desc(j - 2, b).wait()

            @pl.when(j < n_l)
            def _consume():
                g_desc(j, b).wait()
                widen(b)
                s_desc(j, b).start()

        return _

    lax.fori_loop(0, (43 + 2 + 1) // 2, outer, None)


_sc_gather = functools.partial(
    pl.kernel,
    out_type=jax.ShapeDtypeStruct((L, B, O), jnp.float32),
    mesh=plsc.VectorSubcoreMesh(core_axis_name="c", subcore_axis_name="s"),
    compiler_params=pltpu.CompilerParams(needs_layout_passes=False),
    scratch_types=[
        pltpu.VMEM((LG, BC), jnp.int32),
        pltpu.VMEM((BC, O), jnp.float32),
        pltpu.VMEM((BC, O), jnp.float32),
        pltpu.VMEM((BC, O // 2), jnp.int32),
        pltpu.VMEM((BC, O // 2), jnp.int32),
        *([pltpu.SemaphoreType.DMA] * 4),
    ],
)(_sc_body)


def kernel(tile, x, y, tile_table, tile_g, tile_b, col_table, col_g, col_b,
           row_table, row_g, row_b, W, bias):
    c_table, idx4 = _prep(tile, x, y, tile_table, tile_g, tile_b, col_table,
                          col_g, col_b, row_table, row_g, row_b, W, bias)
    out = _sc_gather(idx4, c_table)
    return out.transpose(1, 0, 2)


# speedup vs baseline: 1.0110x; 1.0110x over previous
import functools

import numpy as np

import jax
import jax.numpy as jnp
from jax import lax
from jax.experimental import pallas as pl
from jax.experimental.pallas import tpu as pltpu
from jax.experimental.pallas import tpu_sc as plsc

B, L = 1024, 169
TILE_CLASSES, WIDTH, HEIGHT, H, O = 32, 13, 13, 128, 256
NCOMB = TILE_CLASSES * WIDTH * HEIGHT

NC, NS = 2, 16
NL, NB = 2, 16
BC = B // NB
LG = 88
_L_STARTS = (0, 85)
_L_SIZES = (85, 84)

def _ln_gelu(t, g, b):
    mu = jnp.mean(t, axis=-1, keepdims=True)
    var = jnp.mean((t - mu) ** 2, axis=-1, keepdims=True)
    v = (t - mu) / jnp.sqrt(var + 1e-5) * g + b
    return 0.5 * v * (1.0 + lax.erf(v * (2.0 ** -0.5)))


def _prep_body(tile_ref, x_ref, y_ref, tt_ref, tg_ref, tb_ref, ct_ref, cg_ref,
               cb_ref, rt_ref, rg_ref, rb_ref, w_ref, bias_ref, c_ref, idx_ref):
    f32 = jnp.float32
    pt = jnp.dot(_ln_gelu(tt_ref[...], tg_ref[...], tb_ref[...]),
                 w_ref[0:H, :], preferred_element_type=f32)
    pc = jnp.dot(_ln_gelu(ct_ref[...], cg_ref[...], cb_ref[...]),
                 w_ref[H:2 * H, :], preferred_element_type=f32)
    pr = jnp.dot(_ln_gelu(rt_ref[...], rg_ref[...], rb_ref[...]),
                 w_ref[2 * H:3 * H, :], preferred_element_type=f32)

    def onehot(nrows, ncols, row_to_col):
        r = lax.broadcasted_iota(jnp.int32, (nrows, ncols), 0)
        c = lax.broadcasted_iota(jnp.int32, (nrows, ncols), 1)
        return (row_to_col(r) == c).astype(f32)

    oht = onehot(NCOMB, TILE_CLASSES, lambda r: r // (WIDTH * HEIGHT))
    ohc = onehot(NCOMB, WIDTH, lambda r: (r // HEIGHT) % WIDTH)
    ohr = onehot(NCOMB, HEIGHT, lambda r: r % HEIGHT)
    cc = (jnp.dot(oht, pt, preferred_element_type=f32)
          + jnp.dot(ohc, pc, preferred_element_type=f32)
          + jnp.dot(ohr, pr, preferred_element_type=f32)
          + bias_ref[...])

    def bfbits(v):
        bits = lax.bitcast_convert_type(v, jnp.int32)
        return lax.shift_right_logical(
            bits + 0x7FFF + ((bits >> 16) & 1), 16)

    c_ref[...] = bfbits(cc[:, 0:H]) | (bfbits(cc[:, H:2 * H]) << 16)

    idx_t = jnp.transpose(tile_ref[...] * (WIDTH * HEIGHT)
                          + x_ref[...] * HEIGHT + y_ref[...])
    for g in range(NL):
        ls, n = _L_STARTS[g], _L_SIZES[g]
        idx_ref[g, 0:n, :] = idx_t[ls:ls + n, :]
        idx_ref[g, n:LG, :] = jnp.zeros((LG - n, B), jnp.int32)


def _prep(tile, x, y, tt, tg, tb, ct, cg, cb, rt, rg, rb, w, bias):
    return pl.pallas_call(
        _prep_body,
        out_shape=(
            jax.ShapeDtypeStruct((NCOMB, O // 2), jnp.int32),
            jax.ShapeDtypeStruct((NL, LG, B), jnp.int32),
        ),
    )(tile, x, y, tt, tg.reshape(1, H), tb.reshape(1, H), ct,
      cg.reshape(1, H), cb.reshape(1, H), rt, rg.reshape(1, H),
      rb.reshape(1, H), w, bias.reshape(1, O))


def _sc_body(idx_hbm, c_hbm, out_hbm, idxw, *rest):
    fbufs, bbufs = rest[0:3], rest[3:6]
    gsems, ssems = rest[6:9], rest[9:12]
    wid = lax.axis_index("s") * NC + lax.axis_index("c")
    g = wid // NB
    bb = wid % NB
    b0 = bb * BC
    l_start = jnp.where(g == 0, 0, 85)
    n_l = jnp.where(g == 0, 85, 84)

    pltpu.sync_copy(idx_hbm.at[g, :, pl.ds((bb // 2) * 2 * BC, 2 * BC)], idxw)
    ihalf = (bb % 2) * BC

    def g_desc(j, b):
        return pltpu.make_async_copy(
            c_hbm.at[idxw.at[j, pl.ds(ihalf, BC)]], bbufs[b], gsems[b])

    def s_desc(j, b):
        return pltpu.make_async_copy(
            fbufs[b], out_hbm.at[l_start + j, pl.ds(b0, BC)], ssems[b])

    def widen(b):
        def body(t2, _):
            for dt in range(2):
                t = t2 * 2 + dt
                for k in range(O // 32):
                    xi = bbufs[b][t, pl.ds(16 * k, 16)]
                    lo = plsc.bitcast(xi << 16, jnp.float32)
                    hi = plsc.bitcast(xi & jnp.int32(-65536), jnp.float32)
                    fbufs[b][t, pl.ds(16 * k, 16)] = lo
                    fbufs[b][t, pl.ds(H + 16 * k, 16)] = hi
            return _

        lax.fori_loop(0, BC // 2, body, None)

    g_desc(0, 0).start()
    g_desc(1, 1).start()

    def outer(k, _):
        for b in range(3):
            j = k * 3 + b
            b2 = (b + 2) % 3

            @pl.when(j + 2 < n_l)
            def _prefetch():
                g_desc(j + 2, b2).start()

            @pl.when((j >= 3) & (j - 3 < n_l))
            def _retire():
                s_desc(j - 3, b).wait()

            @pl.when(j < n_l)
            def _consume():
                g_desc(j, b).wait()
                widen(b)
                s_desc(j, b).start()

        return _

    lax.fori_loop(0, (85 + 3 + 2) // 3, outer, None)


_sc_gather = functools.partial(
    pl.kernel,
    out_type=jax.ShapeDtypeStruct((L, B, O), jnp.float32),
    mesh=plsc.VectorSubcoreMesh(core_axis_name="c", subcore_axis_name="s"),
    compiler_params=pltpu.CompilerParams(needs_layout_passes=False),
    scratch_types=[
        pltpu.VMEM((LG, 2 * BC), jnp.int32),
        *([pltpu.VMEM((BC, O), jnp.float32)] * 3),
        *([pltpu.VMEM((BC, O // 2), jnp.int32)] * 3),
        *([pltpu.SemaphoreType.DMA] * 6),
    ],
)(_sc_body)


def kernel(tile, x, y, tile_table, tile_g, tile_b, col_table, col_g, col_b,
           row_table, row_g, row_b, W, bias):
    c_table, idx4 = _prep(tile, x, y, tile_table, tile_g, tile_b, col_table,
                          col_g, col_b, row_table, row_g, row_b, W, bias)
    out = _sc_gather(idx4, c_table)
    return out.transpose(1, 0, 2)


# speedup vs baseline: 1.4761x; 1.4601x over previous
import functools

import jax
import jax.numpy as jnp
from jax import lax
from jax.experimental import pallas as pl
from jax.experimental.pallas import tpu as pltpu
from jax.experimental.pallas import tpu_sc as plsc

B, L = 1024, 169
TILE_CLASSES, WIDTH, HEIGHT, H, O = 32, 13, 13, 128, 256
NCOMB = TILE_CLASSES * WIDTH * HEIGHT

NC, NS = 2, 16
NL, NB = 4, 8
BC = B // NB
LG = 44
NBUF = 3
_L_STARTS = (0, 43, 85, 127)
_L_SIZES = (43, 42, 42, 42)


def _ln_gelu(t, g, b):
    mu = jnp.mean(t, axis=-1, keepdims=True)
    var = jnp.mean((t - mu) ** 2, axis=-1, keepdims=True)
    v = (t - mu) / jnp.sqrt(var + 1e-5) * g + b
    return 0.5 * v * (1.0 + lax.erf(v * (2.0 ** -0.5)))


def _prep_body(tile_ref, x_ref, y_ref, tt_ref, tg_ref, tb_ref, ct_ref, cg_ref,
               cb_ref, rt_ref, rg_ref, rb_ref, w_ref, bias_ref, c_ref, idx_ref):
    f32 = jnp.float32
    pt = jnp.dot(_ln_gelu(tt_ref[...], tg_ref[...], tb_ref[...]),
                 w_ref[0:H, :], preferred_element_type=f32)
    pc = jnp.dot(_ln_gelu(ct_ref[...], cg_ref[...], cb_ref[...]),
                 w_ref[H:2 * H, :], preferred_element_type=f32)
    pr = jnp.dot(_ln_gelu(rt_ref[...], rg_ref[...], rb_ref[...]),
                 w_ref[2 * H:3 * H, :], preferred_element_type=f32)

    def onehot(nrows, ncols, row_to_col):
        r = lax.broadcasted_iota(jnp.int32, (nrows, ncols), 0)
        c = lax.broadcasted_iota(jnp.int32, (nrows, ncols), 1)
        return (row_to_col(r) == c).astype(f32)

    oht = onehot(NCOMB, TILE_CLASSES, lambda r: r // (WIDTH * HEIGHT))
    ohc = onehot(NCOMB, WIDTH, lambda r: (r // HEIGHT) % WIDTH)
    ohr = onehot(NCOMB, HEIGHT, lambda r: r % HEIGHT)
    c_ref[...] = (jnp.dot(oht, pt, preferred_element_type=f32)
                  + jnp.dot(ohc, pc, preferred_element_type=f32)
                  + jnp.dot(ohr, pr, preferred_element_type=f32)
                  + bias_ref[...])

    idx_t = jnp.transpose(tile_ref[...] * (WIDTH * HEIGHT)
                          + x_ref[...] * HEIGHT + y_ref[...])
    for g in range(NL):
        ls, n = _L_STARTS[g], _L_SIZES[g]
        idx_ref[g, 0:n, :] = idx_t[ls:ls + n, :]
        idx_ref[g, n:LG, :] = jnp.zeros((LG - n, B), jnp.int32)


def _prep(tile, x, y, tt, tg, tb, ct, cg, cb, rt, rg, rb, w, bias):
    return pl.pallas_call(
        _prep_body,
        out_shape=(
            jax.ShapeDtypeStruct((NCOMB, O), jnp.float32),
            jax.ShapeDtypeStruct((NL, LG, B), jnp.int32),
        ),
    )(tile, x, y, tt, tg.reshape(1, H), tb.reshape(1, H), ct,
      cg.reshape(1, H), cb.reshape(1, H), rt, rg.reshape(1, H),
      rb.reshape(1, H), w, bias.reshape(1, O))


def _sc_body(idx_hbm, c_hbm, out_hbm, idxw, *rest):
    bufs = rest[0:NBUF]
    gsems = rest[NBUF:2 * NBUF]
    ssems = rest[2 * NBUF:3 * NBUF]
    wid = lax.axis_index("s") * NC + lax.axis_index("c")
    g = wid // NB
    b0 = (wid % NB) * BC
    l_start = jnp.where(g == 0, 0, 43 + (g - 1) * 42)
    n_l = jnp.where(g == 0, 43, 42)

    pltpu.sync_copy(idx_hbm.at[g, :, pl.ds(b0, BC)], idxw)

    def g_desc(j, b):
        return pltpu.make_async_copy(c_hbm.at[idxw.at[j]], bufs[b], gsems[b])

    def s_desc(j, b):
        return pltpu.make_async_copy(
            bufs[b], out_hbm.at[l_start + j, pl.ds(b0, BC)], ssems[b])

    g_desc(0, 0).start()
    g_desc(1, 1).start()

    def outer(k, _):
        for b in range(NBUF):
            j = k * NBUF + b
            b2 = (b + 2) % NBUF

            @pl.when(j < n_l)
            def _consume():
                g_desc(j, b).wait()
                s_desc(j, b).start()

            @pl.when((j >= 1) & (j - 1 < n_l))
            def _retire():
                s_desc(j - 1, b2).wait()

            @pl.when(j + 2 < n_l)
            def _prefetch():
                g_desc(j + 2, b2).start()

        return _

    lax.fori_loop(0, (43 + 1 + NBUF - 1) // NBUF, outer, None)


_sc_gather = functools.partial(
    pl.kernel,
    out_type=jax.ShapeDtypeStruct((L, B, O), jnp.float32),
    mesh=plsc.VectorSubcoreMesh(core_axis_name="c", subcore_axis_name="s"),
    scratch_types=[
        pltpu.VMEM((LG, BC), jnp.int32),
        *([pltpu.VMEM((BC, O), jnp.float32)] * NBUF),
        *([pltpu.SemaphoreType.DMA] * (2 * NBUF)),
    ],
)(_sc_body)


def kernel(tile, x, y, tile_table, tile_g, tile_b, col_table, col_g, col_b,
           row_table, row_g, row_b, W, bias):
    c_table, idx4 = _prep(tile, x, y, tile_table, tile_g, tile_b, col_table,
                          col_g, col_b, row_table, row_g, row_b, W, bias)
    out = _sc_gather(idx4, c_table)
    return out.transpose(1, 0, 2)
